# Initial kernel scaffold; baseline (speedup 1.0000x reference)
import jax
import jax.numpy as jnp
from jax.experimental import pallas as pl


def kernel(x, edge_index, Wl1, bl1, Wr1, Wl2, bl2, Wr2, Wl3, bl3, Wr3, Wc, bc):
    raise NotImplementedError("write your pallas kernel here")


# speedup vs baseline: 4.6374x; 4.6374x over previous
import jax
import jax.numpy as jnp
from jax import lax
from jax.experimental import pallas as pl
from jax.experimental.pallas import tpu as pltpu
from jax.experimental.pallas import tpu_sc as plsc

N = 10000
E = 320000
NC = 2
NS = 16
EW = E // NS
CH = 128
NFULL = EW // CH
TAIL = EW - NFULL * CH
NP = 10240
ZR = NP // NS
CW = 16
ROWBLK = 1000


def _sc_pass(ph, src, dst, with_count):
    d2 = ph.shape[2]
    mesh = plsc.VectorSubcoreMesh(
        core_axis_name="c", subcore_axis_name="s", num_cores=NC,
        num_subcores=NS)

    out_type = [jax.ShapeDtypeStruct((NC, NP, d2), jnp.float32)]
    if with_count:
        out_type.append(jax.ShapeDtypeStruct((NP, CW), jnp.float32))

    scratch = dict(
        idx_s=pltpu.VMEM((CH,), jnp.int32),
        idx_d=pltpu.VMEM((CH,), jnp.int32),
        idx_s_t=pltpu.VMEM((TAIL,), jnp.int32),
        idx_d_t=pltpu.VMEM((TAIL,), jnp.int32),
        rows=pltpu.VMEM((CH, d2), jnp.float32),
        rows_t=pltpu.VMEM((TAIL, d2), jnp.float32),
        zbuf=pltpu.VMEM((ZR, d2), jnp.float32),
        acc=pltpu.VMEM_SHARED((NP, d2), jnp.float32),
        sem=pltpu.SemaphoreType.DMA,
    )
    if with_count:
        scratch.update(
            ones_v=pltpu.VMEM((CH, CW), jnp.float32),
            ones_t=pltpu.VMEM((TAIL, CW), jnp.float32),
            czbuf=pltpu.VMEM((ZR, CW), jnp.float32),
            cacc=pltpu.VMEM_SHARED((NP, CW), jnp.float32),
        )

    def body(ph_hbm, src_hbm, dst_hbm, *outs, **scr):
        idx_s, idx_d, idx_s_t, idx_d_t = (scr["idx_s"], scr["idx_d"],
                                          scr["idx_s_t"], scr["idx_d_t"])
        rows, rows_t, zbuf, acc, sem = (scr["rows"], scr["rows_t"],
                                        scr["zbuf"], scr["acc"], scr["sem"])
        out_hbm = outs[0]
        cid = lax.axis_index("c")
        sid = lax.axis_index("s")
        is_c0 = cid == 0

        zeros16 = jnp.zeros((16,), jnp.float32)
        ones16 = jnp.ones((16,), jnp.float32)
        dl = d2 // 16

        def zfill(i, _):
            zbuf[i // dl, pl.ds((i % dl) * 16, 16)] = zeros16
            return 0
        lax.fori_loop(0, ZR * dl, zfill, 0)
        pltpu.sync_copy(zbuf, acc.at[pl.ds(sid * ZR, ZR)])

        if with_count:
            @pl.when(is_c0)
            def _():
                czbuf, cacc = scr["czbuf"], scr["cacc"]
                ones_v, ones_t = scr["ones_v"], scr["ones_t"]

                def czfill(i, _):
                    czbuf[i, pl.ds(0, 16)] = zeros16
                    return 0
                lax.fori_loop(0, ZR, czfill, 0)

                def ofill(i, _):
                    ones_v[i, pl.ds(0, 16)] = ones16
                    return 0
                lax.fori_loop(0, CH, ofill, 0)

                def otfill(i, _):
                    ones_t[i, pl.ds(0, 16)] = ones16
                    return 0
                lax.fori_loop(0, TAIL, otfill, 0)
                pltpu.sync_copy(czbuf, cacc.at[pl.ds(sid * ZR, ZR)])

        plsc.subcore_barrier()

        base = sid * EW

        def chunk(i, _):
            off = base + i * CH
            pltpu.sync_copy(src_hbm.at[pl.ds(off, CH)], idx_s)
            pltpu.sync_copy(dst_hbm.at[pl.ds(off, CH)], idx_d)
            pltpu.async_copy(ph_hbm.at[cid].at[idx_s], rows, sem).wait()
            pltpu.sync_copy(rows, acc.at[idx_d], add=True)
            if with_count:
                @pl.when(is_c0)
                def _():
                    pltpu.sync_copy(scr["ones_v"], scr["cacc"].at[idx_d],
                                    add=True)
            return 0
        lax.fori_loop(0, NFULL, chunk, 0)

        off = base + NFULL * CH
        pltpu.sync_copy(src_hbm.at[pl.ds(off, TAIL)], idx_s_t)
        pltpu.sync_copy(dst_hbm.at[pl.ds(off, TAIL)], idx_d_t)
        pltpu.async_copy(ph_hbm.at[cid].at[idx_s_t], rows_t, sem).wait()
        pltpu.sync_copy(rows_t, acc.at[idx_d_t], add=True)
        if with_count:
            @pl.when(is_c0)
            def _():
                pltpu.sync_copy(scr["ones_t"], scr["cacc"].at[idx_d_t],
                                add=True)

        plsc.subcore_barrier()

        pltpu.sync_copy(acc.at[pl.ds(sid * ZR, ZR)],
                        out_hbm.at[cid, pl.ds(sid * ZR, ZR)])
        if with_count:
            @pl.when(is_c0)
            def _():
                pltpu.sync_copy(scr["cacc"].at[pl.ds(sid * ZR, ZR)],
                                outs[1].at[pl.ds(sid * ZR, ZR)])

    fn = pl.kernel(body, out_type=tuple(out_type), mesh=mesh,
                   scratch_types=scratch,
                   compiler_params=pltpu.CompilerParams(
                       use_tc_tiling_on_sc=False))
    return fn(ph, src, dst)


def _tc_pre(x, wt):
    n, k = x.shape
    m = wt.shape[1]
    m2 = m // 2

    def body(x_ref, w_ref, o_ref):
        p = jnp.dot(x_ref[...], w_ref[...],
                    preferred_element_type=jnp.float32)
        o_ref[0] = p[:, :m2]
        o_ref[1] = p[:, m2:]

    return pl.pallas_call(
        body,
        grid=(n // ROWBLK,),
        in_specs=[
            pl.BlockSpec((ROWBLK, k), lambda i: (i, 0)),
            pl.BlockSpec((k, m), lambda i: (0, 0)),
        ],
        out_specs=pl.BlockSpec((NC, ROWBLK, m2), lambda i: (0, i, 0)),
        out_shape=jax.ShapeDtypeStruct((NC, n, m2), jnp.float32),
    )(x, wt)


def _tc_post(agg, cnt, h_in, wrt, bl, wlnt):
    n, d_in = h_in.shape
    d = wrt.shape[1]
    d2 = d // 2
    dn = wlnt.shape[1]
    dn2 = dn // 2

    def body(a_ref, c_ref, h_ref, wr_ref, bl_ref, wl_ref, ho_ref, po_ref):
        c = c_ref[:, 0:1]
        inv = 1.0 / jnp.maximum(c, 1.0)
        root = jnp.dot(h_ref[...], wr_ref[...],
                       preferred_element_type=jnp.float32)
        a = jnp.concatenate([a_ref[0], a_ref[1]], axis=1)
        h = jnp.maximum(a * inv + bl_ref[...] + root, 0.0)
        ho_ref[...] = h
        p = jnp.dot(h, wl_ref[...], preferred_element_type=jnp.float32)
        po_ref[0] = p[:, :dn2]
        po_ref[1] = p[:, dn2:]

    return pl.pallas_call(
        body,
        grid=(n // ROWBLK,),
        in_specs=[
            pl.BlockSpec((NC, ROWBLK, d2), lambda i: (0, i, 0)),
            pl.BlockSpec((ROWBLK, CW), lambda i: (i, 0)),
            pl.BlockSpec((ROWBLK, d_in), lambda i: (i, 0)),
            pl.BlockSpec((d_in, d), lambda i: (0, 0)),
            pl.BlockSpec((1, d), lambda i: (0, 0)),
            pl.BlockSpec((d, dn), lambda i: (0, 0)),
        ],
        out_specs=[
            pl.BlockSpec((ROWBLK, d), lambda i: (i, 0)),
            pl.BlockSpec((NC, ROWBLK, dn2), lambda i: (0, i, 0)),
        ],
        out_shape=[
            jax.ShapeDtypeStruct((n, d), jnp.float32),
            jax.ShapeDtypeStruct((NC, n, dn2), jnp.float32),
        ],
    )(agg, cnt, h_in, wrt, bl, wlnt)


def _tc_final(agg, cnt, h_in, wrt, bl, wct, bc):
    n, d_in = h_in.shape
    d = wrt.shape[1]
    d2 = d // 2
    m = wct.shape[1]

    def body(a_ref, c_ref, h_ref, wr_ref, bl_ref, wc_ref, bc_ref, o_ref):
        c = c_ref[:, 0:1]
        inv = 1.0 / jnp.maximum(c, 1.0)
        root = jnp.dot(h_ref[...], wr_ref[...],
                       preferred_element_type=jnp.float32)
        a = jnp.concatenate([a_ref[0], a_ref[1]], axis=1)
        h = jnp.maximum(a * inv + bl_ref[...] + root, 0.0)
        o_ref[...] = jnp.dot(h, wc_ref[...],
                             preferred_element_type=jnp.float32) + bc_ref[...]

    return pl.pallas_call(
        body,
        grid=(n // ROWBLK,),
        in_specs=[
            pl.BlockSpec((NC, ROWBLK, d2), lambda i: (0, i, 0)),
            pl.BlockSpec((ROWBLK, CW), lambda i: (i, 0)),
            pl.BlockSpec((ROWBLK, d_in), lambda i: (i, 0)),
            pl.BlockSpec((d_in, d), lambda i: (0, 0)),
            pl.BlockSpec((1, d), lambda i: (0, 0)),
            pl.BlockSpec((d, m), lambda i: (0, 0)),
            pl.BlockSpec((1, m), lambda i: (0, 0)),
        ],
        out_specs=pl.BlockSpec((ROWBLK, m), lambda i: (i, 0)),
        out_shape=jax.ShapeDtypeStruct((n, m), jnp.float32),
    )(agg, cnt, h_in, wrt, bl, wct, bc)


@jax.jit
def kernel(x, edge_index, Wl1, bl1, Wr1, Wl2, bl2, Wr2, Wl3, bl3, Wr3,
           Wc, bc):
    src = edge_index[0]
    dst = edge_index[1]

    p1 = _tc_pre(x, Wl1.T)
    agg1, cnt = _sc_pass(p1, src, dst, True)
    h1, p2 = _tc_post(agg1, cnt, x, Wr1.T, bl1.reshape(1, -1), Wl2.T)
    agg2, = _sc_pass(p2, src, dst, False)
    h2, p3 = _tc_post(agg2, cnt, h1, Wr2.T, bl2.reshape(1, -1), Wl3.T)
    agg3, = _sc_pass(p3, src, dst, False)
    return _tc_final(agg3, cnt, h2, Wr3.T, bl3.reshape(1, -1), Wc.T,
                     bc.reshape(1, -1))


# speedup vs baseline: 7.5828x; 1.6351x over previous
import jax
import jax.numpy as jnp
from jax import lax
from jax.experimental import pallas as pl
from jax.experimental.pallas import tpu as pltpu
from jax.experimental.pallas import tpu_sc as plsc

N = 10000
E = 320000
NC = 2
NS = 16
CH = 128
CR = 158
HCR = CR // 2
EROWS = NS * CR
EPAD = EROWS * CH
NP = 10240
DUMP = NP - 2
ZR = NP // NS
ZC = 64
CW = 16
ROWBLK = 1000


def _sc_pass(ph, src2d, dst2d, with_count):
    d2 = ph.shape[2]
    mesh = plsc.VectorSubcoreMesh(
        core_axis_name="c", subcore_axis_name="s", num_cores=NC,
        num_subcores=NS)

    out_type = [jax.ShapeDtypeStruct((NC, NP, d2), jnp.float32)]
    if with_count:
        out_type.append(jax.ShapeDtypeStruct((NP, CW), jnp.float32))

    scratch = dict(
        src_buf=pltpu.VMEM((CR, CH), jnp.int32),
        dst_buf=pltpu.VMEM((CR, CH), jnp.int32),
        rows0=pltpu.VMEM((CH, d2), jnp.float32),
        rows1=pltpu.VMEM((CH, d2), jnp.float32),
        zbuf=pltpu.VMEM((ZC, d2), jnp.float32),
        acc=pltpu.VMEM_SHARED((NP, d2), jnp.float32),
        sem0=pltpu.SemaphoreType.DMA,
        sem1=pltpu.SemaphoreType.DMA,
    )
    if with_count:
        scratch.update(
            ones_v=pltpu.VMEM((CH, CW), jnp.float32),
            czbuf=pltpu.VMEM((ZC, CW), jnp.float32),
            cacc=pltpu.VMEM_SHARED((NP, CW), jnp.float32),
        )

    def body(ph_hbm, src_hbm, dst_hbm, *outs, **scr):
        src_buf, dst_buf = scr["src_buf"], scr["dst_buf"]
        rows = (scr["rows0"], scr["rows1"])
        sems = (scr["sem0"], scr["sem1"])
        zbuf, acc = scr["zbuf"], scr["acc"]
        out_hbm = outs[0]
        cid = lax.axis_index("c")
        sid = lax.axis_index("s")
        is_c0 = cid == 0
        tbl = ph_hbm.at[cid]

        pltpu.sync_copy(src_hbm.at[pl.ds(sid * CR, CR)], src_buf)
        pltpu.sync_copy(dst_hbm.at[pl.ds(sid * CR, CR)], dst_buf)

        zeros16 = jnp.zeros((16,), jnp.float32)
        ones16 = jnp.ones((16,), jnp.float32)
        dl = d2 // 16

        def zfill(i, _):
            zbuf[i // dl, pl.ds((i % dl) * 16, 16)] = zeros16
            return 0
        lax.fori_loop(0, ZC * dl, zfill, 0)

        def zcopy(j, _):
            pltpu.sync_copy(zbuf, acc.at[pl.ds(sid * ZR + j * ZC, ZC)])
            return 0
        lax.fori_loop(0, ZR // ZC, zcopy, 0)

        if with_count:
            czbuf, cacc, ones_v = scr["czbuf"], scr["cacc"], scr["ones_v"]

            def czfill(i, _):
                czbuf[i, pl.ds(0, 16)] = zeros16
                return 0
            lax.fori_loop(0, ZC, czfill, 0)

            def ofill(i, _):
                ones_v[i, pl.ds(0, 16)] = ones16
                return 0
            lax.fori_loop(0, CH, ofill, 0)

            def czcopy(j, _):
                pltpu.sync_copy(czbuf, cacc.at[pl.ds(sid * ZR + j * ZC, ZC)])
                return 0
            lax.fori_loop(0, ZR // ZC, czcopy, 0)

        pltpu.async_copy(tbl.at[src_buf.at[0]], rows[0], sems[0])
        plsc.subcore_barrier()

        def count_scatter(c):
            @pl.when(is_c0)
            def _():
                pltpu.sync_copy(ones_v, cacc.at[dst_buf.at[c]], add=True)

        def step(i, _):
            c0 = i * 2
            c1 = c0 + 1
            pltpu.make_async_copy(tbl.at[src_buf.at[c0]], rows[0],
                                  sems[0]).wait()
            pltpu.async_copy(tbl.at[src_buf.at[c1]], rows[1], sems[1])
            pltpu.sync_copy(rows[0], acc.at[dst_buf.at[c0]], add=True)
            if with_count:
                count_scatter(c0)
            pltpu.make_async_copy(tbl.at[src_buf.at[c1]], rows[1],
                                  sems[1]).wait()

            @pl.when(i < HCR - 1)
            def _():
                pltpu.async_copy(tbl.at[src_buf.at[c0 + 2]], rows[0],
                                 sems[0])
            pltpu.sync_copy(rows[1], acc.at[dst_buf.at[c1]], add=True)
            if with_count:
                count_scatter(c1)
            return 0
        lax.fori_loop(0, HCR, step, 0)

        plsc.subcore_barrier()

        pltpu.sync_copy(acc.at[pl.ds(sid * ZR, ZR)],
                        out_hbm.at[cid, pl.ds(sid * ZR, ZR)])
        if with_count:
            @pl.when(is_c0)
            def _():
                pltpu.sync_copy(cacc.at[pl.ds(sid * ZR, ZR)],
                                outs[1].at[pl.ds(sid * ZR, ZR)])

    fn = pl.kernel(body, out_type=tuple(out_type), mesh=mesh,
                   scratch_types=scratch,
                   compiler_params=pltpu.CompilerParams(
                       use_tc_tiling_on_sc=False))
    return fn(ph, src2d, dst2d)


def _tc_pre(x, wt):
    n, k = x.shape
    m = wt.shape[1]
    m2 = m // 2

    def body(x_ref, w_ref, o_ref):
        p = jnp.dot(x_ref[...], w_ref[...],
                    preferred_element_type=jnp.float32)
        o_ref[0] = p[:, :m2]
        o_ref[1] = p[:, m2:]

    return pl.pallas_call(
        body,
        grid=(n // ROWBLK,),
        in_specs=[
            pl.BlockSpec((ROWBLK, k), lambda i: (i, 0)),
            pl.BlockSpec((k, m), lambda i: (0, 0)),
        ],
        out_specs=pl.BlockSpec((NC, ROWBLK, m2), lambda i: (0, i, 0)),
        out_shape=jax.ShapeDtypeStruct((NC, n, m2), jnp.float32),
    )(x, wt)


def _tc_post(agg, cnt, h_in, wrt, bl, wlnt):
    n, d_in = h_in.shape
    d = wrt.shape[1]
    d2 = d // 2
    dn = wlnt.shape[1]
    dn2 = dn // 2

    def body(a_ref, c_ref, h_ref, wr_ref, bl_ref, wl_ref, ho_ref, po_ref):
        c = c_ref[:, 0:1]
        inv = 1.0 / jnp.maximum(c, 1.0)
        root = jnp.dot(h_ref[...], wr_ref[...],
                       preferred_element_type=jnp.float32)
        a = jnp.concatenate([a_ref[0], a_ref[1]], axis=1)
        h = jnp.maximum(a * inv + bl_ref[...] + root, 0.0)
        ho_ref[...] = h
        p = jnp.dot(h, wl_ref[...], preferred_element_type=jnp.float32)
        po_ref[0] = p[:, :dn2]
        po_ref[1] = p[:, dn2:]

    return pl.pallas_call(
        body,
        grid=(n // ROWBLK,),
        in_specs=[
            pl.BlockSpec((NC, ROWBLK, d2), lambda i: (0, i, 0)),
            pl.BlockSpec((ROWBLK, CW), lambda i: (i, 0)),
            pl.BlockSpec((ROWBLK, d_in), lambda i: (i, 0)),
            pl.BlockSpec((d_in, d), lambda i: (0, 0)),
            pl.BlockSpec((1, d), lambda i: (0, 0)),
            pl.BlockSpec((d, dn), lambda i: (0, 0)),
        ],
        out_specs=[
            pl.BlockSpec((ROWBLK, d), lambda i: (i, 0)),
            pl.BlockSpec((NC, ROWBLK, dn2), lambda i: (0, i, 0)),
        ],
        out_shape=[
            jax.ShapeDtypeStruct((n, d), jnp.float32),
            jax.ShapeDtypeStruct((NC, n, dn2), jnp.float32),
        ],
    )(agg, cnt, h_in, wrt, bl, wlnt)


def _tc_final(agg, cnt, h_in, wrt, bl, wct, bc):
    n, d_in = h_in.shape
    d = wrt.shape[1]
    d2 = d // 2
    m = wct.shape[1]

    def body(a_ref, c_ref, h_ref, wr_ref, bl_ref, wc_ref, bc_ref, o_ref):
        c = c_ref[:, 0:1]
        inv = 1.0 / jnp.maximum(c, 1.0)
        root = jnp.dot(h_ref[...], wr_ref[...],
                       preferred_element_type=jnp.float32)
        a = jnp.concatenate([a_ref[0], a_ref[1]], axis=1)
        h = jnp.maximum(a * inv + bl_ref[...] + root, 0.0)
        o_ref[...] = jnp.dot(h, wc_ref[...],
                             preferred_element_type=jnp.float32) + bc_ref[...]

    return pl.pallas_call(
        body,
        grid=(n // ROWBLK,),
        in_specs=[
            pl.BlockSpec((NC, ROWBLK, d2), lambda i: (0, i, 0)),
            pl.BlockSpec((ROWBLK, CW), lambda i: (i, 0)),
            pl.BlockSpec((ROWBLK, d_in), lambda i: (i, 0)),
            pl.BlockSpec((d_in, d), lambda i: (0, 0)),
            pl.BlockSpec((1, d), lambda i: (0, 0)),
            pl.BlockSpec((d, m), lambda i: (0, 0)),
            pl.BlockSpec((1, m), lambda i: (0, 0)),
        ],
        out_specs=pl.BlockSpec((ROWBLK, m), lambda i: (i, 0)),
        out_shape=jax.ShapeDtypeStruct((n, m), jnp.float32),
    )(agg, cnt, h_in, wrt, bl, wct, bc)


@jax.jit
def kernel(x, edge_index, Wl1, bl1, Wr1, Wl2, bl2, Wr2, Wl3, bl3, Wr3,
           Wc, bc):
    pad_s = jnp.zeros((EPAD - E,), jnp.int32)
    pad_d = jnp.full((EPAD - E,), DUMP, jnp.int32)
    src2d = jnp.concatenate([edge_index[0], pad_s]).reshape(EROWS, CH)
    dst2d = jnp.concatenate([edge_index[1], pad_d]).reshape(EROWS, CH)

    p1 = _tc_pre(x, Wl1.T)
    agg1, cnt = _sc_pass(p1, src2d, dst2d, True)
    h1, p2 = _tc_post(agg1, cnt, x, Wr1.T, bl1.reshape(1, -1), Wl2.T)
    agg2, = _sc_pass(p2, src2d, dst2d, False)
    h2, p3 = _tc_post(agg2, cnt, h1, Wr2.T, bl2.reshape(1, -1), Wl3.T)
    agg3, = _sc_pass(p3, src2d, dst2d, False)
    return _tc_final(agg3, cnt, h2, Wr3.T, bl3.reshape(1, -1), Wc.T,
                     bc.reshape(1, -1))
